# baseline (speedup 1.0000x reference)
import functools
import numpy as np
import jax
import jax.numpy as jnp
from jax import lax
from jax.experimental import pallas as pl
from jax.experimental.pallas import tpu as pltpu
from jax.experimental.pallas import tpu_sc as plsc

VOCAB = 100000
LATENT = 64
NUM_EV = 200
WORDS = 50

NC = 2
NS = 16
S_PER = 8
N_SENT_TILES = NUM_EV // S_PER
ROWS_PER_TILE = S_PER * WORDS
QPAD = 64


def _position_encoding() -> np.ndarray:
    i = np.arange(WORDS, dtype=np.float32)[:, None]
    j = np.arange(LATENT, dtype=np.float32)[None, :]
    return (1.0 - i / WORDS - (j / LATENT) * (1.0 - 2.0 * i / WORDS)).astype(np.float32)


def _sc_embed(ev_idx, q_idx, q_table, e_table, t1, t2, pe,
              evc_out, evf_out, q_out,
              idx_v, rows_q, rows_e, pe_v, t1_v, t2_v, evc_v, evf_v, qv,
              sem1, sem2):
    wid = lax.axis_index("s") * NC + lax.axis_index("c")
    zero = jnp.zeros((16,), jnp.float32)

    @pl.when(wid < N_SENT_TILES)
    def _():
        base = wid * S_PER
        pltpu.sync_copy(ev_idx.at[pl.ds(base * WORDS, ROWS_PER_TILE)], idx_v)
        cp1 = pltpu.async_copy(q_table.at[idx_v], rows_q, sem1)
        cp2 = pltpu.async_copy(e_table.at[idx_v], rows_e, sem2)
        pltpu.sync_copy(pe, pe_v)
        pltpu.sync_copy(t1.at[pl.ds(base, S_PER)], t1_v)
        pltpu.sync_copy(t2.at[pl.ds(base, S_PER)], t2_v)
        cp1.wait()
        cp2.wait()
        for s in range(S_PER):
            for ci in range(LATENT // 16):
                col = pl.ds(ci * 16, 16)

                def body(w, carry, s=s, col=col):
                    aq, ae = carry
                    p = pe_v[w, col]
                    aq = aq + rows_q[s * WORDS + w, col] * p
                    ae = ae + rows_e[s * WORDS + w, col] * p
                    return (aq, ae)

                aq, ae = lax.fori_loop(0, WORDS, body, (zero, zero))
                evc_v[s, col] = aq + t2_v[s, col]
                evf_v[s, col] = ae + t1_v[s, col]
        pltpu.sync_copy(evc_v, evc_out.at[pl.ds(base, S_PER)])
        pltpu.sync_copy(evf_v, evf_out.at[pl.ds(base, S_PER)])

    @pl.when(wid == N_SENT_TILES)
    def _():
        pltpu.sync_copy(q_idx, idx_v.at[pl.ds(0, QPAD)])
        pltpu.async_copy(q_table.at[idx_v.at[pl.ds(0, QPAD)]],
                         rows_q.at[pl.ds(0, QPAD)], sem1).wait()
        pltpu.sync_copy(pe, pe_v)
        for ci in range(LATENT // 16):
            col = pl.ds(ci * 16, 16)

            def qbody(w, acc, col=col):
                return acc + rows_q[w, col] * pe_v[w, col]

            acc = lax.fori_loop(0, WORDS, qbody, zero)
            qv[col] = acc
        pltpu.sync_copy(qv, q_out)


@jax.jit
def _sc_call(ev_idx, q_idx, q_table, e_table, t1, t2, pe):
    mesh = plsc.VectorSubcoreMesh(core_axis_name="c", subcore_axis_name="s",
                                  num_cores=NC, num_subcores=NS)
    f32 = jnp.float32
    return pl.kernel(
        _sc_embed,
        out_type=(
            jax.ShapeDtypeStruct((NUM_EV, LATENT), f32),
            jax.ShapeDtypeStruct((NUM_EV, LATENT), f32),
            jax.ShapeDtypeStruct((LATENT,), f32),
        ),
        mesh=mesh,
        scratch_types=(
            pltpu.VMEM((ROWS_PER_TILE,), jnp.int32),
            pltpu.VMEM((ROWS_PER_TILE, LATENT), f32),
            pltpu.VMEM((ROWS_PER_TILE, LATENT), f32),
            pltpu.VMEM((WORDS, LATENT), f32),
            pltpu.VMEM((S_PER, LATENT), f32),
            pltpu.VMEM((S_PER, LATENT), f32),
            pltpu.VMEM((S_PER, LATENT), f32),
            pltpu.VMEM((S_PER, LATENT), f32),
            pltpu.VMEM((LATENT,), f32),
            pltpu.SemaphoreType.DMA,
            pltpu.SemaphoreType.DMA,
        ),
        compiler_params=pltpu.CompilerParams(use_tc_tiling_on_sc=False),
    )(ev_idx, q_idx, q_table, e_table, t1, t2, pe)


BV = 4000
NB = VOCAB // BV


def _tc_body(evc_ref, evf_ref, q_ref, fcw_ref, fcb_ref, out_ref,
             logit_s, feat_s, ms_s):
    ph = pl.program_id(0)
    j = pl.program_id(1)

    @pl.when((ph == 0) & (j == 0))
    def _():
        q = q_ref[...]
        z = lax.dot_general(evc_ref[...], q, (((1,), (1,)), ((), ())),
                            preferred_element_type=jnp.float32)
        z = z - jnp.max(z)
        e = jnp.exp(z)
        w = e / jnp.sum(e)
        pooled = lax.dot_general(w, evf_ref[...], (((0,), (0,)), ((), ())),
                                 preferred_element_type=jnp.float32)
        feat_s[...] = pooled + q
        ms_s[0] = -jnp.inf
        ms_s[1] = 0.0

    @pl.when(ph == 0)
    def _():
        f = feat_s[...]
        l = lax.dot_general(f, fcw_ref[...], (((1,), (1,)), ((), ())),
                            preferred_element_type=jnp.float32)
        l = l + fcb_ref[0]
        logit_s[pl.ds(j, 1), :] = l
        m_old = ms_s[0]
        m_new = jnp.maximum(m_old, jnp.max(l))
        ms_s[1] = ms_s[1] * jnp.exp(m_old - m_new) + jnp.sum(jnp.exp(l - m_new))
        ms_s[0] = m_new

    @pl.when(ph == 1)
    def _():
        l = logit_s[pl.ds(j, 1), :]
        out_ref[0] = jnp.exp(l - ms_s[0]) * (1.0 / ms_s[1])


@jax.jit
def _tc_call(evc, evf, q2, fc_w, fc_b2):
    f32 = jnp.float32
    return pl.pallas_call(
        _tc_body,
        grid=(2, NB),
        in_specs=[
            pl.BlockSpec((NUM_EV, LATENT), lambda p, j: (0, 0)),
            pl.BlockSpec((NUM_EV, LATENT), lambda p, j: (0, 0)),
            pl.BlockSpec((1, LATENT), lambda p, j: (0, 0)),
            pl.BlockSpec((BV, LATENT), lambda p, j: (j * (1 - p), 0)),
            pl.BlockSpec((1, 1, BV), lambda p, j: (j * (1 - p), 0, 0)),
        ],
        out_specs=pl.BlockSpec((1, 1, BV), lambda p, j: (j, 0, 0)),
        out_shape=jax.ShapeDtypeStruct((NB, 1, BV), f32),
        scratch_shapes=[
            pltpu.VMEM((NB, BV), f32),
            pltpu.VMEM((1, LATENT), f32),
            pltpu.SMEM((2,), f32),
        ],
        compiler_params=pltpu.CompilerParams(
            dimension_semantics=("arbitrary", "arbitrary"),
        ),
    )(evc, evf, q2, fc_w, fc_b2)


def kernel(evidence, question, question_table, evidence_table,
           temporal_enc1, temporal_enc2, fc_w, fc_b):
    pe = jnp.asarray(_position_encoding())
    ev_idx = evidence.reshape(-1).astype(jnp.int32)
    q_flat = question.reshape(-1).astype(jnp.int32)
    q_idx = jnp.zeros((QPAD,), jnp.int32).at[:WORDS].set(q_flat)

    evc, evf, q = _sc_call(ev_idx, q_idx, question_table, evidence_table,
                           temporal_enc1, temporal_enc2, pe)
    probs = _tc_call(evc, evf, q.reshape(1, LATENT), fc_w,
                     fc_b.reshape(NB, 1, BV))
    return probs.reshape(VOCAB)


# speedup vs baseline: 3.0901x; 3.0901x over previous
import functools
import numpy as np
import jax
import jax.numpy as jnp
from jax import lax
from jax.experimental import pallas as pl
from jax.experimental.pallas import tpu as pltpu
from jax.experimental.pallas import tpu_sc as plsc

VOCAB = 100000
LATENT = 64
NUM_EV = 200
WORDS = 50

NC = 2
NS = 16
NW = NC * NS
J_PER = LATENT // NW
NSENT = 208
NGRP = NSENT // 16


def _sc_embed(tq, te, idx_flat, evc_out, evf_out, row_v, idx_v, out_v, sem):
    wid = lax.axis_index("s") * NC + lax.axis_index("c")
    zero = jnp.zeros((16,), jnp.float32)
    inv_w = jnp.float32(1.0 / WORDS)
    inv_l = jnp.float32(1.0 / LATENT)
    pltpu.sync_copy(idx_flat, idx_v)

    for tbl, out, jo in ((tq, evc_out, 0), (tq, evc_out, 1),
                         (te, evf_out, 0), (te, evf_out, 1)):
        j = wid * J_PER + jo
        pltpu.async_copy(tbl.at[j], row_v, sem).wait()
        jf = jnp.full((16,), j, jnp.float32) * inv_l

        def body(w, accs, jf=jf):
            wf = jnp.full((16,), w, jnp.float32) * inv_w
            pe = (1.0 - wf) + jf * (2.0 * wf - 1.0)
            new = []
            for g in range(NGRP):
                iv = idx_v[pl.ds(w * NSENT + g * 16, 16)]
                vals = plsc.load_gather(row_v, [iv])
                new.append(accs[g] + vals * pe)
            return tuple(new)

        accs = lax.fori_loop(0, WORDS, body, (zero,) * NGRP)
        for g in range(NGRP):
            out_v[pl.ds(g * 16, 16)] = accs[g]
        pltpu.sync_copy(out_v, out.at[j])


@jax.jit
def _sc_call(tq, te, idx_flat):
    mesh = plsc.VectorSubcoreMesh(core_axis_name="c", subcore_axis_name="s",
                                  num_cores=NC, num_subcores=NS)
    f32 = jnp.float32
    return pl.kernel(
        _sc_embed,
        out_type=(
            jax.ShapeDtypeStruct((LATENT, NSENT), f32),
            jax.ShapeDtypeStruct((LATENT, NSENT), f32),
        ),
        mesh=mesh,
        scratch_types=(
            pltpu.VMEM((VOCAB,), f32),
            pltpu.VMEM((WORDS * NSENT,), jnp.int32),
            pltpu.VMEM((NSENT,), f32),
            pltpu.SemaphoreType.DMA,
        ),
        compiler_params=pltpu.CompilerParams(use_tc_tiling_on_sc=True,
                                             needs_layout_passes=False),
    )(tq, te, idx_flat)


BVC = 8192
NBLK = (VOCAB + BVC - 1) // BVC


def _tc_body(evc_ref, evf_ref, t1_ref, t2_ref, fcw_ref, fcb_ref, out_ref,
             logit_s, feat_s, ms_s):
    ph = pl.program_id(0)
    j = pl.program_id(1)
    dnum_00 = (((0,), (0,)), ((), ()))
    dnum_11 = (((1,), (1,)), ((), ()))

    @pl.when((ph == 0) & (j == 0))
    def _():
        evc = evc_ref[...]
        evc_e = evc + t2_ref[...]
        evf_e = evf_ref[...] + t1_ref[...]
        lane = lax.broadcasted_iota(jnp.int32, (1, NSENT), 1)
        zf = lax.dot_general(evc, evc_e, dnum_00,
                             preferred_element_type=jnp.float32)
        z = zf[NUM_EV:NUM_EV + 1, :]
        z = jnp.where(lane < NUM_EV, z, -jnp.inf)
        z = z - jnp.max(z)
        e = jnp.exp(z)
        w = e / jnp.sum(e)
        onehot = (lane == NUM_EV).astype(jnp.float32)
        feat_s[...] = (
            lax.dot_general(evf_e, w, dnum_11,
                            preferred_element_type=jnp.float32)
            + lax.dot_general(evc, onehot, dnum_11,
                              preferred_element_type=jnp.float32))
        ms_s[0] = -jnp.inf
        ms_s[1] = 0.0

    @pl.when(ph == 0)
    def _():
        f = feat_s[...]
        l = lax.dot_general(f, fcw_ref[...], dnum_00,
                            preferred_element_type=jnp.float32)
        l = l + fcb_ref[0]
        col = j * BVC + lax.broadcasted_iota(jnp.int32, (1, BVC), 1)
        valid = col < VOCAB
        l = jnp.where(valid, l, -jnp.inf)
        logit_s[pl.ds(j, 1), :] = l
        m_old = ms_s[0]
        m_new = jnp.maximum(m_old, jnp.max(l))
        contrib = jnp.sum(jnp.where(valid, jnp.exp(l - m_new), 0.0))
        ms_s[1] = ms_s[1] * jnp.exp(m_old - m_new) + contrib
        ms_s[0] = m_new

    @pl.when(ph == 1)
    def _():
        l = logit_s[pl.ds(j, 1), :]
        out_ref[0] = jnp.exp(l - ms_s[0]) * (1.0 / ms_s[1])


@jax.jit
def _tc_call(evcT, evfT, t1T, t2T, fc_wT, fc_b):
    f32 = jnp.float32
    return pl.pallas_call(
        _tc_body,
        grid=(2, NBLK),
        in_specs=[
            pl.BlockSpec((LATENT, NSENT), lambda p, j: (0, 0)),
            pl.BlockSpec((LATENT, NSENT), lambda p, j: (0, 0)),
            pl.BlockSpec((LATENT, NSENT), lambda p, j: (0, 0)),
            pl.BlockSpec((LATENT, NSENT), lambda p, j: (0, 0)),
            pl.BlockSpec((LATENT, BVC), lambda p, j: (0, j * (1 - p))),
            pl.BlockSpec((1, 1, BVC), lambda p, j: (j * (1 - p), 0, 0)),
        ],
        out_specs=pl.BlockSpec((1, 1, BVC), lambda p, j: (j, 0, 0)),
        out_shape=jax.ShapeDtypeStruct((NBLK, 1, BVC), f32),
        scratch_shapes=[
            pltpu.VMEM((NBLK, BVC), f32),
            pltpu.VMEM((LATENT, 1), f32),
            pltpu.SMEM((2,), f32),
        ],
        compiler_params=pltpu.CompilerParams(
            dimension_semantics=("arbitrary", "arbitrary"),
        ),
    )(evcT, evfT, t1T, t2T, fc_wT, fc_b)


def kernel(evidence, question, question_table, evidence_table,
           temporal_enc1, temporal_enc2, fc_w, fc_b):
    ev_T = evidence.astype(jnp.int32).T
    q_T = question.astype(jnp.int32).T
    pad = jnp.zeros((WORDS, NSENT - NUM_EV - 1), jnp.int32)
    idx_flat = jnp.concatenate([ev_T, q_T, pad], axis=1).reshape(-1)

    t1p = jnp.pad(temporal_enc1.T, ((0, 0), (0, NSENT - NUM_EV)))
    t2p = jnp.pad(temporal_enc2.T, ((0, 0), (0, NSENT - NUM_EV)))
    fcb2 = jnp.pad(fc_b, (0, NBLK * BVC - VOCAB)).reshape(NBLK, 1, BVC)

    evcT, evfT = _sc_call(question_table.T, evidence_table.T, idx_flat)
    probs2 = _tc_call(evcT, evfT, t1p, t2p, fc_w.T, fcb2)
    return probs2.reshape(-1)[:VOCAB]


# speedup vs baseline: 3.2909x; 1.0650x over previous
import functools
import numpy as np
import jax
import jax.numpy as jnp
from jax import lax
from jax.experimental import pallas as pl
from jax.experimental.pallas import tpu as pltpu
from jax.experimental.pallas import tpu_sc as plsc

VOCAB = 100000
LATENT = 64
NUM_EV = 200
WORDS = 50

NC = 2
NS = 16
NW = NC * NS
J_PER = LATENT // NW
NSENT = 208
NGRP = NSENT // 16


def _sc_embed(tq, te, idx_flat, evc_out, evf_out, row_v, idx_v, out_v, sem):
    wid = lax.axis_index("s") * NC + lax.axis_index("c")
    zero = jnp.zeros((16,), jnp.float32)
    inv_w = jnp.float32(1.0 / WORDS)
    inv_l = jnp.float32(1.0 / LATENT)
    pltpu.sync_copy(idx_flat, idx_v)

    for tbl, out, jo in ((tq, evc_out, 0), (tq, evc_out, 1),
                         (te, evf_out, 0), (te, evf_out, 1)):
        j = wid * J_PER + jo
        pltpu.async_copy(tbl.at[j], row_v, sem).wait()
        jf = jnp.full((16,), j, jnp.float32) * inv_l

        def body(w, accs, jf=jf):
            wf = jnp.full((16,), w, jnp.float32) * inv_w
            pe = (1.0 - wf) + jf * (2.0 * wf - 1.0)
            new = []
            for g in range(NGRP):
                iv = idx_v[pl.ds(w * NSENT + g * 16, 16)]
                vals = plsc.load_gather(row_v, [iv])
                new.append(accs[g] + vals * pe)
            return tuple(new)

        accs = lax.fori_loop(0, WORDS, body, (zero,) * NGRP)
        for g in range(NGRP):
            out_v[pl.ds(g * 16, 16)] = accs[g]
        pltpu.sync_copy(out_v, out.at[j])


@jax.jit
def _sc_call(tq, te, idx_flat):
    mesh = plsc.VectorSubcoreMesh(core_axis_name="c", subcore_axis_name="s",
                                  num_cores=NC, num_subcores=NS)
    f32 = jnp.float32
    return pl.kernel(
        _sc_embed,
        out_type=(
            jax.ShapeDtypeStruct((LATENT, NSENT), f32),
            jax.ShapeDtypeStruct((LATENT, NSENT), f32),
        ),
        mesh=mesh,
        scratch_types=(
            pltpu.VMEM((VOCAB,), f32),
            pltpu.VMEM((WORDS * NSENT,), jnp.int32),
            pltpu.VMEM((NSENT,), f32),
            pltpu.SemaphoreType.DMA,
        ),
        compiler_params=pltpu.CompilerParams(use_tc_tiling_on_sc=True,
                                             needs_layout_passes=False),
    )(tq, te, idx_flat)


BVC = 8192
NBLK = (VOCAB + BVC - 1) // BVC


def _tc_body(evc_ref, evf_ref, t1_ref, t2_ref, fcw_ref, fcb_ref, out_ref,
             logit_s, feat_s, ms_s):
    j = pl.program_id(0)
    dnum_00 = (((0,), (0,)), ((), ()))
    dnum_11 = (((1,), (1,)), ((), ()))

    @pl.when(j == 0)
    def _():
        evc = evc_ref[...]
        evc_e = evc + t2_ref[...]
        evf_e = evf_ref[...] + t1_ref[...]
        lane = lax.broadcasted_iota(jnp.int32, (1, NSENT), 1)
        zf = lax.dot_general(evc, evc_e, dnum_00,
                             preferred_element_type=jnp.float32)
        z = zf[NUM_EV:NUM_EV + 1, :]
        z = jnp.where(lane < NUM_EV, z, -jnp.inf)
        z = z - jnp.max(z)
        e = jnp.exp(z)
        w = e / jnp.sum(e)
        onehot = (lane == NUM_EV).astype(jnp.float32)
        feat_s[...] = (
            lax.dot_general(evf_e, w, dnum_11,
                            preferred_element_type=jnp.float32)
            + lax.dot_general(evc, onehot, dnum_11,
                              preferred_element_type=jnp.float32))
        ms_s[0] = -jnp.inf
        ms_s[1] = 0.0

    @pl.when(j < NBLK)
    def _():
        f = feat_s[...]
        l = lax.dot_general(f, fcw_ref[...], dnum_00,
                            preferred_element_type=jnp.float32)
        l = l + fcb_ref[0]
        col = j * BVC + lax.broadcasted_iota(jnp.int32, (1, BVC), 1)
        valid = col < VOCAB
        l = jnp.where(valid, l, -jnp.inf)
        logit_s[pl.ds(j, 1), :] = l
        m_old = ms_s[0]
        m_new = jnp.maximum(m_old, jnp.max(l))
        contrib = jnp.sum(jnp.where(valid, jnp.exp(l - m_new), 0.0))
        ms_s[1] = ms_s[1] * jnp.exp(m_old - m_new) + contrib
        ms_s[0] = m_new

    @pl.when(j == NBLK)
    def _():
        probs = jnp.exp(logit_s[...] - ms_s[0]) * (1.0 / ms_s[1])
        out_ref[...] = probs.reshape(NBLK, 1, BVC)


@jax.jit
def _tc_call(evcT, evfT, t1T, t2T, fc_wT, fc_b):
    f32 = jnp.float32
    return pl.pallas_call(
        _tc_body,
        grid=(NBLK + 1,),
        in_specs=[
            pl.BlockSpec((LATENT, NSENT), lambda j: (0, 0)),
            pl.BlockSpec((LATENT, NSENT), lambda j: (0, 0)),
            pl.BlockSpec((LATENT, NSENT), lambda j: (0, 0)),
            pl.BlockSpec((LATENT, NSENT), lambda j: (0, 0)),
            pl.BlockSpec((LATENT, BVC),
                         lambda j: (0, jnp.minimum(j, NBLK - 1))),
            pl.BlockSpec((1, 1, BVC),
                         lambda j: (jnp.minimum(j, NBLK - 1), 0, 0)),
        ],
        out_specs=pl.BlockSpec((NBLK, 1, BVC), lambda j: (0, 0, 0)),
        out_shape=jax.ShapeDtypeStruct((NBLK, 1, BVC), f32),
        scratch_shapes=[
            pltpu.VMEM((NBLK, BVC), f32),
            pltpu.VMEM((LATENT, 1), f32),
            pltpu.SMEM((2,), f32),
        ],
        compiler_params=pltpu.CompilerParams(
            dimension_semantics=("arbitrary",),
        ),
    )(evcT, evfT, t1T, t2T, fc_wT, fc_b)


def kernel(evidence, question, question_table, evidence_table,
           temporal_enc1, temporal_enc2, fc_w, fc_b):
    ev_T = evidence.astype(jnp.int32).T
    q_T = question.astype(jnp.int32).T
    pad = jnp.zeros((WORDS, NSENT - NUM_EV - 1), jnp.int32)
    idx_flat = jnp.concatenate([ev_T, q_T, pad], axis=1).reshape(-1)

    t1p = jnp.pad(temporal_enc1.T, ((0, 0), (0, NSENT - NUM_EV)))
    t2p = jnp.pad(temporal_enc2.T, ((0, 0), (0, NSENT - NUM_EV)))
    fcb2 = jnp.pad(fc_b, (0, NBLK * BVC - VOCAB)).reshape(NBLK, 1, BVC)

    evcT, evfT = _sc_call(question_table.T, evidence_table.T, idx_flat)
    probs2 = _tc_call(evcT, evfT, t1p, t2p, fc_w.T, fcb2)
    return probs2.reshape(-1)[:VOCAB]


# speedup vs baseline: 3.3767x; 1.0261x over previous
import functools
import numpy as np
import jax
import jax.numpy as jnp
from jax import lax
from jax.experimental import pallas as pl
from jax.experimental.pallas import tpu as pltpu
from jax.experimental.pallas import tpu_sc as plsc

VOCAB = 100000
LATENT = 64
NUM_EV = 200
WORDS = 50

NC = 2
NS = 16
NW = NC * NS
J_PER = LATENT // NW
NSENT = 208
NGRP = NSENT // 16


def _sc_embed(tq, te, idx_flat, evc_out, evf_out, row_v, idx_v, out_v, sem):
    wid = lax.axis_index("s") * NC + lax.axis_index("c")
    zero = jnp.zeros((16,), jnp.float32)
    inv_w = jnp.float32(1.0 / WORDS)
    inv_l = jnp.float32(1.0 / LATENT)
    pltpu.sync_copy(idx_flat, idx_v)

    for tbl, out, jo in ((tq, evc_out, 0), (tq, evc_out, 1),
                         (te, evf_out, 0), (te, evf_out, 1)):
        j = wid * J_PER + jo
        pltpu.async_copy(tbl.at[j], row_v, sem).wait()
        jf = jnp.full((16,), j, jnp.float32) * inv_l

        def body(w, accs, jf=jf):
            wf = jnp.full((16,), w, jnp.float32) * inv_w
            pe = (1.0 - wf) + jf * (2.0 * wf - 1.0)
            new = []
            for g in range(NGRP):
                iv = idx_v[pl.ds(w * NSENT + g * 16, 16)]
                vals = plsc.load_gather(row_v, [iv])
                new.append(accs[g] + vals * pe)
            return tuple(new)

        accs = lax.fori_loop(0, WORDS, body, (zero,) * NGRP)
        for g in range(NGRP):
            out_v[pl.ds(g * 16, 16)] = accs[g]
        pltpu.sync_copy(out_v, out.at[j])


@jax.jit
def _sc_call(tq, te, idx_flat):
    mesh = plsc.VectorSubcoreMesh(core_axis_name="c", subcore_axis_name="s",
                                  num_cores=NC, num_subcores=NS)
    f32 = jnp.float32
    return pl.kernel(
        _sc_embed,
        out_type=(
            jax.ShapeDtypeStruct((LATENT, NSENT), f32),
            jax.ShapeDtypeStruct((LATENT, NSENT), f32),
        ),
        mesh=mesh,
        scratch_types=(
            pltpu.VMEM((VOCAB,), f32),
            pltpu.VMEM((WORDS * NSENT,), jnp.int32),
            pltpu.VMEM((NSENT,), f32),
            pltpu.SemaphoreType.DMA,
        ),
        compiler_params=pltpu.CompilerParams(use_tc_tiling_on_sc=True,
                                             needs_layout_passes=False),
    )(tq, te, idx_flat)


BVC = 8192
NBLK = (VOCAB + BVC - 1) // BVC


def _tc_body(evc_ref, evf_ref, t1_ref, t2_ref, fcw_ref, fcb_ref, out_ref,
             logit_s, feat_s, ms_s):
    j = pl.program_id(0)
    dnum_00 = (((0,), (0,)), ((), ()))
    dnum_11 = (((1,), (1,)), ((), ()))

    @pl.when(j == 0)
    def _():
        evc = evc_ref[...]
        evc_e = evc + t2_ref[...]
        evf_e = evf_ref[...] + t1_ref[...]
        lane = lax.broadcasted_iota(jnp.int32, (1, NSENT), 1)
        zf = lax.dot_general(evc, evc_e, dnum_00,
                             preferred_element_type=jnp.float32)
        z = zf[NUM_EV:NUM_EV + 1, :]
        z = jnp.where(lane < NUM_EV, z, -jnp.inf)
        z = z - jnp.max(z)
        e = jnp.exp(z)
        w = e / jnp.sum(e)
        onehot = (lane == NUM_EV).astype(jnp.float32)
        feat_s[...] = (
            lax.dot_general(w, evf_e, dnum_11,
                            preferred_element_type=jnp.float32)
            + lax.dot_general(onehot, evc, dnum_11,
                              preferred_element_type=jnp.float32))
        ms_s[0] = -jnp.inf

    @pl.when(j < NBLK)
    def _():
        f = feat_s[...]
        l = lax.dot_general(f, fcw_ref[...], (((1,), (0,)), ((), ())),
                            preferred_element_type=jnp.float32)
        l = l + fcb_ref[0]
        col = j * BVC + lax.broadcasted_iota(jnp.int32, (1, BVC), 1)
        l = jnp.where(col < VOCAB, l, -1e30)
        logit_s[pl.ds(j, 1), :] = l
        ms_s[0] = jnp.maximum(ms_s[0], jnp.max(l))

    @pl.when(j == NBLK)
    def _():
        e = jnp.exp(logit_s[...] - ms_s[0])
        out_ref[...] = (e * (1.0 / jnp.sum(e))).reshape(NBLK, 1, BVC)


@jax.jit
def _tc_call(evcT, evfT, t1T, t2T, fc_wT, fc_b):
    f32 = jnp.float32
    return pl.pallas_call(
        _tc_body,
        grid=(NBLK + 1,),
        in_specs=[
            pl.BlockSpec((LATENT, NSENT), lambda j: (0, 0)),
            pl.BlockSpec((LATENT, NSENT), lambda j: (0, 0)),
            pl.BlockSpec((LATENT, NSENT), lambda j: (0, 0)),
            pl.BlockSpec((LATENT, NSENT), lambda j: (0, 0)),
            pl.BlockSpec((LATENT, BVC),
                         lambda j: (0, jnp.minimum(j, NBLK - 1))),
            pl.BlockSpec((1, 1, BVC),
                         lambda j: (jnp.minimum(j, NBLK - 1), 0, 0)),
        ],
        out_specs=pl.BlockSpec((NBLK, 1, BVC), lambda j: (0, 0, 0)),
        out_shape=jax.ShapeDtypeStruct((NBLK, 1, BVC), f32),
        scratch_shapes=[
            pltpu.VMEM((NBLK, BVC), f32),
            pltpu.VMEM((1, LATENT), f32),
            pltpu.SMEM((2,), f32),
        ],
        compiler_params=pltpu.CompilerParams(
            dimension_semantics=("arbitrary",),
        ),
    )(evcT, evfT, t1T, t2T, fc_wT, fc_b)


def kernel(evidence, question, question_table, evidence_table,
           temporal_enc1, temporal_enc2, fc_w, fc_b):
    ev_T = evidence.astype(jnp.int32).T
    q_T = question.astype(jnp.int32).T
    pad = jnp.zeros((WORDS, NSENT - NUM_EV - 1), jnp.int32)
    idx_flat = jnp.concatenate([ev_T, q_T, pad], axis=1).reshape(-1)

    t1p = jnp.pad(temporal_enc1.T, ((0, 0), (0, NSENT - NUM_EV)))
    t2p = jnp.pad(temporal_enc2.T, ((0, 0), (0, NSENT - NUM_EV)))
    fcb2 = jnp.pad(fc_b, (0, NBLK * BVC - VOCAB),
                   constant_values=-1e30).reshape(NBLK, 1, BVC)

    evcT, evfT = _sc_call(question_table.T, evidence_table.T, idx_flat)
    probs2 = _tc_call(evcT, evfT, t1p, t2p, fc_w.T, fcb2)
    return probs2.reshape(-1)[:VOCAB]


# speedup vs baseline: 3.3836x; 1.0020x over previous
import functools
import numpy as np
import jax
import jax.numpy as jnp
from jax import lax
from jax.experimental import pallas as pl
from jax.experimental.pallas import tpu as pltpu
from jax.experimental.pallas import tpu_sc as plsc

VOCAB = 100000
LATENT = 64
NUM_EV = 200
WORDS = 50

NC = 2
NS = 16
NW = NC * NS
J_PER = LATENT // NW
NSENT = 208
NGRP = NSENT // 16


def _sc_embed(tq, te, idxT, evc_out, evf_out, row_v, idx_v,
              ov0, ov1, ov2, ov3, sem, semo):
    wid = lax.axis_index("s") * NC + lax.axis_index("c")
    zero = jnp.zeros((16,), jnp.float32)
    inv_w = jnp.float32(1.0 / WORDS)
    inv_l = jnp.float32(1.0 / LATENT)

    passes = ((tq, evc_out, 0, ov0), (tq, evc_out, 1, ov1),
              (te, evf_out, 0, ov2), (te, evf_out, 1, ov3))
    cp = pltpu.async_copy(tq.at[wid * J_PER], row_v, sem)
    pltpu.sync_copy(idxT, idx_v)

    out_cps = []
    for p, (tbl, out, jo, ov) in enumerate(passes):
        j = wid * J_PER + jo
        cp.wait()
        jf = jnp.full((16,), j, jnp.float32) * inv_l

        def body(w, accs, jf=jf):
            wf = jnp.full((16,), w, jnp.float32) * inv_w
            pe = (1.0 - wf) + jf * (2.0 * wf - 1.0)
            new = []
            for g in range(NGRP):
                iv = idx_v[w, pl.ds(g * 16, 16)]
                vals = plsc.load_gather(row_v, [iv])
                new.append(accs[g] + vals * pe)
            return tuple(new)

        accs = lax.fori_loop(0, WORDS, body, (zero,) * NGRP)
        if p < 3:
            ntbl, _, njo, _ = passes[p + 1]
            cp = pltpu.async_copy(ntbl.at[wid * J_PER + njo], row_v, sem)
        for g in range(NGRP):
            ov[pl.ds(g * 16, 16)] = accs[g]
        out_cps.append(pltpu.async_copy(ov, out.at[j], semo))
    for c in out_cps:
        c.wait()


@jax.jit
def _sc_call(tq, te, idxT):
    mesh = plsc.VectorSubcoreMesh(core_axis_name="c", subcore_axis_name="s",
                                  num_cores=NC, num_subcores=NS)
    f32 = jnp.float32
    return pl.kernel(
        _sc_embed,
        out_type=(
            jax.ShapeDtypeStruct((LATENT, NSENT), f32),
            jax.ShapeDtypeStruct((LATENT, NSENT), f32),
        ),
        mesh=mesh,
        scratch_types=(
            pltpu.VMEM((VOCAB,), f32),
            pltpu.VMEM((WORDS, NSENT), jnp.int32),
            pltpu.VMEM((NSENT,), f32),
            pltpu.VMEM((NSENT,), f32),
            pltpu.VMEM((NSENT,), f32),
            pltpu.VMEM((NSENT,), f32),
            pltpu.SemaphoreType.DMA,
            pltpu.SemaphoreType.DMA,
        ),
        compiler_params=pltpu.CompilerParams(use_tc_tiling_on_sc=True,
                                             needs_layout_passes=False),
    )(tq, te, idxT)


BVC = 8192
NBLK = (VOCAB + BVC - 1) // BVC


def _tc_body(evc_ref, evf_ref, t1_ref, t2_ref, fcw_ref, fcb_ref, out_ref,
             logit_s, feat_s, ms_s):
    j = pl.program_id(0)
    dnum_00 = (((0,), (0,)), ((), ()))
    dnum_11 = (((1,), (1,)), ((), ()))

    @pl.when(j == 0)
    def _():
        evc = evc_ref[...]
        evc_e = evc + t2_ref[...]
        evf_e = evf_ref[...] + t1_ref[...]
        lane = lax.broadcasted_iota(jnp.int32, (1, NSENT), 1)
        zf = lax.dot_general(evc, evc_e, dnum_00,
                             preferred_element_type=jnp.float32)
        z = zf[NUM_EV:NUM_EV + 1, :]
        z = jnp.where(lane < NUM_EV, z, -jnp.inf)
        z = z - jnp.max(z)
        e = jnp.exp(z)
        w = e / jnp.sum(e)
        onehot = (lane == NUM_EV).astype(jnp.float32)
        feat_s[...] = (
            lax.dot_general(w, evf_e, dnum_11,
                            preferred_element_type=jnp.float32)
            + lax.dot_general(onehot, evc, dnum_11,
                              preferred_element_type=jnp.float32))
        ms_s[0] = -jnp.inf

    @pl.when(j < NBLK)
    def _():
        f = feat_s[...]
        l = lax.dot_general(f, fcw_ref[...], (((1,), (0,)), ((), ())),
                            preferred_element_type=jnp.float32)
        l = l + fcb_ref[0]
        col = j * BVC + lax.broadcasted_iota(jnp.int32, (1, BVC), 1)
        l = jnp.where(col < VOCAB, l, -1e30)
        logit_s[pl.ds(j, 1), :] = l
        ms_s[0] = jnp.maximum(ms_s[0], jnp.max(l))

    @pl.when(j == NBLK)
    def _():
        e = jnp.exp(logit_s[...] - ms_s[0])
        out_ref[...] = (e * (1.0 / jnp.sum(e))).reshape(NBLK, 1, BVC)


@jax.jit
def _tc_call(evcT, evfT, t1T, t2T, fc_wT, fc_b):
    f32 = jnp.float32
    return pl.pallas_call(
        _tc_body,
        grid=(NBLK + 1,),
        in_specs=[
            pl.BlockSpec((LATENT, NSENT), lambda j: (0, 0)),
            pl.BlockSpec((LATENT, NSENT), lambda j: (0, 0)),
            pl.BlockSpec((LATENT, NSENT), lambda j: (0, 0)),
            pl.BlockSpec((LATENT, NSENT), lambda j: (0, 0)),
            pl.BlockSpec((LATENT, BVC),
                         lambda j: (0, jnp.minimum(j, NBLK - 1))),
            pl.BlockSpec((1, 1, BVC),
                         lambda j: (jnp.minimum(j, NBLK - 1), 0, 0)),
        ],
        out_specs=pl.BlockSpec((NBLK, 1, BVC), lambda j: (0, 0, 0)),
        out_shape=jax.ShapeDtypeStruct((NBLK, 1, BVC), f32),
        scratch_shapes=[
            pltpu.VMEM((NBLK, BVC), f32),
            pltpu.VMEM((1, LATENT), f32),
            pltpu.SMEM((2,), f32),
        ],
        compiler_params=pltpu.CompilerParams(
            dimension_semantics=("arbitrary",),
        ),
    )(evcT, evfT, t1T, t2T, fc_wT, fc_b)


def kernel(evidence, question, question_table, evidence_table,
           temporal_enc1, temporal_enc2, fc_w, fc_b):
    ev_T = evidence.astype(jnp.int32).T
    q_T = question.astype(jnp.int32).T
    pad = jnp.zeros((WORDS, NSENT - NUM_EV - 1), jnp.int32)
    idxT = jnp.concatenate([ev_T, q_T, pad], axis=1)

    t1p = jnp.pad(temporal_enc1.T, ((0, 0), (0, NSENT - NUM_EV)))
    t2p = jnp.pad(temporal_enc2.T, ((0, 0), (0, NSENT - NUM_EV)))
    fcb2 = jnp.pad(fc_b, (0, NBLK * BVC - VOCAB),
                   constant_values=-1e30).reshape(NBLK, 1, BVC)

    evcT, evfT = _sc_call(question_table.T, evidence_table.T, idxT)
    probs2 = _tc_call(evcT, evfT, t1p, t2p, fc_w.T, fcb2)
    return probs2.reshape(-1)[:VOCAB]
